# baseline (device time: 42246 ns/iter reference)
import jax
import jax.numpy as jnp
from jax import lax
from jax.experimental import pallas as pl
from jax.experimental.pallas import tpu as pltpu

W = 16

import os
_DIAG = os.environ.get("KERNEL_DIAG", "full")
_KV_SPACE = (pltpu.VMEM if os.environ.get("KERNEL_KV", "vmem") == "vmem"
             else pl.ANY)


def kernel(Q, K, V):
    b, sq, h, d = Q.shape
    skv = K.shape[1]
    kl = skv // (W // 2)
    scale = d ** -0.5

    def body(q_ref, k_hbm, v_hbm, o_ref,
             k_l, v_l, send_buf, recv_bufs, local_sems, send_sems, recv_sems):
        my_x = lax.axis_index("x")
        my_y = lax.axis_index("y")
        my_z = lax.axis_index("z")
        fid = my_x * 8 + my_y * 4 + my_z
        gid = my_x * 4 + my_z

        if _DIAG != "nodma":
            cp_k = pltpu.make_async_copy(
                k_hbm.at[:, pl.ds(gid * kl, kl)], k_l, local_sems.at[0])
            cp_v = pltpu.make_async_copy(
                v_hbm.at[:, pl.ds(gid * kl, kl)], v_l, local_sems.at[1])
            cp_k.start()
            cp_v.start()

        if _DIAG not in ("local", "nocompute", "nodma"):
            barrier_sem = pltpu.get_barrier_semaphore()
            for off in range(1, W):
                pfid = (fid + off) % W
                pl.semaphore_signal(
                    barrier_sem, inc=1,
                    device_id=(pfid // 8, (pfid // 4) % 2, pfid % 4),
                    device_id_type=pl.DeviceIdType.MESH,
                )
            pl.semaphore_wait(barrier_sem, W - 1)

        if _DIAG != "nodma":
            cp_k.wait()
            cp_v.wait()

        q = q_ref[:, 0, :, :]
        k = k_l[...]
        v = v_l[...]

        if _DIAG == "nocompute":
            l_c = jnp.sum(k[:, :, :, 0], axis=1) + jnp.sum(v[:, :, :, 0], axis=1)
            o_c = q
        else:
            s = jnp.sum(q[:, None, :, :] * k, axis=-1) * scale
            p = jnp.exp(s)
            l_c = jnp.sum(p, axis=1)
            o_c = jnp.sum(p[..., None] * v, axis=1)

        send_buf[0, :, :, :] = o_c
        send_buf[1, :, :, :] = jnp.broadcast_to(l_c[:, :, None], (b, h, d))

        sends = []
        if _DIAG == "full":
            for off in range(1, W):
                pfid = (fid + off) % W
                rd = pltpu.make_async_remote_copy(
                    src_ref=send_buf,
                    dst_ref=recv_bufs.at[fid],
                    send_sem=send_sems.at[pfid],
                    recv_sem=recv_sems.at[fid],
                    device_id=(pfid // 8, (pfid // 4) % 2, pfid % 4),
                    device_id_type=pl.DeviceIdType.MESH,
                )
                rd.start()
                sends.append(rd)
        cp_self = pltpu.make_async_copy(
            send_buf, recv_bufs.at[fid], local_sems.at[2])
        cp_self.start()

        if _DIAG == "full":
            for off in range(1, W):
                pfid = (fid + off) % W
                pltpu.make_async_remote_copy(
                    src_ref=send_buf,
                    dst_ref=recv_bufs.at[pfid],
                    send_sem=send_sems.at[pfid],
                    recv_sem=recv_sems.at[pfid],
                    device_id=(pfid // 8, (pfid // 4) % 2, pfid % 4),
                    device_id_type=pl.DeviceIdType.MESH,
                ).wait_recv()
        cp_self.wait()

        tot = jnp.sum(recv_bufs[...], axis=0)
        o_ref[:, 0, :, :] = tot[0] / tot[1]

        for rd in sends:
            rd.wait_send()

    return pl.pallas_call(
        body,
        out_shape=jax.ShapeDtypeStruct((b, sq, h, d), jnp.float32),
        in_specs=[
            pl.BlockSpec(memory_space=pltpu.VMEM),
            pl.BlockSpec(memory_space=_KV_SPACE),
            pl.BlockSpec(memory_space=_KV_SPACE),
        ],
        out_specs=pl.BlockSpec(memory_space=pltpu.VMEM),
        scratch_shapes=[
            pltpu.VMEM((b, kl, h, d), jnp.float32),
            pltpu.VMEM((b, kl, h, d), jnp.float32),
            pltpu.VMEM((2, b, h, d), jnp.float32),
            pltpu.VMEM((W, 2, b, h, d), jnp.float32),
            pltpu.SemaphoreType.DMA((3,)),
            pltpu.SemaphoreType.DMA((W,)),
            pltpu.SemaphoreType.DMA((W,)),
        ],
        compiler_params=pltpu.CompilerParams(
            collective_id=None if _DIAG in ("local", "nocompute", "nodma") else 0),
    )(Q, K, V)


# device time: 14144 ns/iter; 2.9868x vs baseline; 2.9868x over previous
import jax
import jax.numpy as jnp
from jax import lax
from jax.experimental import pallas as pl
from jax.experimental.pallas import tpu as pltpu


def kernel(Q, K, V):
    b, sq, h, d = Q.shape
    skv = K.shape[1]
    scale = d ** -0.5

    Kt = jnp.transpose(K, (0, 2, 3, 1))
    Vt = jnp.transpose(V, (0, 2, 3, 1))

    def body(q_ref, kt_ref, vt_ref, o_ref,
             send_buf, recv_buf, send_sem, recv_sem):
        my_x = lax.axis_index("x")
        my_y = lax.axis_index("y")
        my_z = lax.axis_index("z")
        partner = (my_x, 1 - my_y, my_z)

        barrier_sem = pltpu.get_barrier_semaphore()
        pl.semaphore_signal(
            barrier_sem, inc=1, device_id=partner,
            device_id_type=pl.DeviceIdType.MESH,
        )
        pl.semaphore_wait(barrier_sem, 1)

        q = q_ref[:, 0, :, :]
        kt = kt_ref[...]
        vt = vt_ref[...]

        s = jnp.sum(q[..., None] * kt, axis=2) * scale
        p = jnp.exp(s)
        l_c = jnp.sum(p, axis=-1)
        o_c = jnp.sum(p[:, :, None, :] * vt, axis=-1)

        send_buf[0, :, :, :] = o_c
        send_buf[1, :, :, :] = jnp.broadcast_to(l_c[:, :, None], (b, h, d))

        rdma = pltpu.make_async_remote_copy(
            src_ref=send_buf,
            dst_ref=recv_buf,
            send_sem=send_sem,
            recv_sem=recv_sem,
            device_id=partner,
            device_id_type=pl.DeviceIdType.MESH,
        )
        rdma.start()
        rdma.wait()

        o_tot = send_buf[0, :, :, :] + recv_buf[0, :, :, :]
        l_tot = send_buf[1, :, :, :] + recv_buf[1, :, :, :]
        o_ref[:, 0, :, :] = o_tot / l_tot

    return pl.pallas_call(
        body,
        out_shape=jax.ShapeDtypeStruct((b, sq, h, d), jnp.float32),
        in_specs=[
            pl.BlockSpec(memory_space=pltpu.VMEM),
            pl.BlockSpec(memory_space=pltpu.VMEM),
            pl.BlockSpec(memory_space=pltpu.VMEM),
        ],
        out_specs=pl.BlockSpec(memory_space=pltpu.VMEM),
        scratch_shapes=[
            pltpu.VMEM((2, b, h, d), jnp.float32),
            pltpu.VMEM((2, b, h, d), jnp.float32),
            pltpu.SemaphoreType.DMA,
            pltpu.SemaphoreType.DMA,
        ],
        compiler_params=pltpu.CompilerParams(collective_id=0),
    )(Q, Kt, Vt)
